# initial kernel scaffold (unmeasured)
import jax
import jax.numpy as jnp
from jax import lax
from jax.experimental import pallas as pl
from jax.experimental.pallas import tpu as pltpu

E = 16
E_LOC = 8
T = 2048
T_LOC = 1024
D = 1024
F = 4096
CAP = 384
FT = 1024

_MESH = pl.DeviceIdType.MESH if hasattr(pl, "DeviceIdType") else pltpu.DeviceIdType.MESH
_CompilerParams = getattr(pltpu, "CompilerParams", None) or pltpu.TPUCompilerParams


def _partner():
    return (1 - lax.axis_index("x"), lax.axis_index("y"), lax.axis_index("z"))


def _partner_barrier():
    barrier = pltpu.get_barrier_semaphore()
    pl.semaphore_signal(barrier, inc=1, device_id=_partner(), device_id_type=_MESH)
    pl.semaphore_wait(barrier, 1)


def _pairwise_exchange(arrays, collective_id):
    n = len(arrays)

    def body(*refs):
        ins = refs[:n]
        outs = refs[n : 2 * n]
        send_sems = refs[2 * n]
        recv_sems = refs[2 * n + 1]
        _partner_barrier()
        rdmas = []
        for k in range(n):
            r = pltpu.make_async_remote_copy(
                src_ref=ins[k],
                dst_ref=outs[k],
                send_sem=send_sems.at[k],
                recv_sem=recv_sems.at[k],
                device_id=_partner(),
                device_id_type=_MESH,
            )
            r.start()
            rdmas.append(r)
        for r in rdmas:
            r.wait()

    out = pl.pallas_call(
        body,
        out_shape=[jax.ShapeDtypeStruct(a.shape, a.dtype) for a in arrays],
        in_specs=[pl.BlockSpec(memory_space=pltpu.VMEM)] * n,
        out_specs=[pl.BlockSpec(memory_space=pltpu.VMEM)] * n,
        scratch_shapes=[
            pltpu.SemaphoreType.DMA((n,)),
            pltpu.SemaphoreType.DMA((n,)),
        ],
        compiler_params=_CompilerParams(collective_id=collective_id),
    )(*arrays)
    return out


def _exchange_add(mine, theirs, collective_id):

    def body(mine_ref, theirs_ref, out_ref, recv_buf, send_sem, recv_sem):
        _partner_barrier()
        r = pltpu.make_async_remote_copy(
            src_ref=theirs_ref,
            dst_ref=recv_buf,
            send_sem=send_sem,
            recv_sem=recv_sem,
            device_id=_partner(),
            device_id_type=_MESH,
        )
        r.start()
        r.wait()
        out_ref[...] = mine_ref[...] + recv_buf[...]

    return pl.pallas_call(
        body,
        out_shape=jax.ShapeDtypeStruct(mine.shape, mine.dtype),
        in_specs=[pl.BlockSpec(memory_space=pltpu.VMEM)] * 2,
        out_specs=pl.BlockSpec(memory_space=pltpu.VMEM),
        scratch_shapes=[
            pltpu.VMEM(mine.shape, mine.dtype),
            pltpu.SemaphoreType.DMA,
            pltpu.SemaphoreType.DMA,
        ],
        compiler_params=_CompilerParams(collective_id=collective_id),
    )(mine, theirs)


def _ffn(Xg, W1, W2):
    h = jnp.einsum(
        "ecd,edf->ecf",
        Xg,
        W1.astype(jnp.bfloat16),
        preferred_element_type=jnp.float32,
    )
    h = jnp.maximum(h, 0.0).astype(jnp.bfloat16)
    return jnp.einsum(
        "ecf,efd->ecd",
        h,
        W2.astype(jnp.bfloat16),
        preferred_element_type=jnp.float32,
    )


def kernel(x, router, W1, W2):
    my_x = lax.axis_index("x")

    def cat(a, b, axis):
        return jnp.where(
            my_x == 0,
            jnp.concatenate([a, b], axis=axis),
            jnp.concatenate([b, a], axis=axis),
        )

    (router_other,) = _pairwise_exchange([router], collective_id=0)
    router_full = cat(router, router_other, axis=1)

    gates_loc = jnp.dot(x, router_full, precision=lax.Precision.HIGHEST)
    x_bf = x.astype(jnp.bfloat16)
    x_other, gates_other = _pairwise_exchange([x_bf, gates_loc], collective_id=1)
    x_full = cat(x_bf, x_other, axis=0)
    gates = cat(gates_loc, gates_other, axis=0)

    top_v, top_i = lax.top_k(gates, 2)
    w = jnp.exp(top_v - top_v[:, 0:1])
    w = w / w.sum(axis=1, keepdims=True)
    w_dense = jnp.zeros((T, E), jnp.float32).at[jnp.arange(T)[:, None], top_i].add(w)
    w_loc = lax.dynamic_slice(w_dense, (0, my_x * E_LOC), (T, E_LOC))

    chosen = w_loc > 0
    order = jnp.argsort(~chosen, axis=0, stable=True)
    idx = order[:CAP, :].T
    Xg = x_full[idx]
    Wg = jnp.take_along_axis(w_loc, idx.T, axis=0).T

    Yg = _ffn(Xg, W1, W2)
    partial = jnp.zeros((T, D), jnp.float32).at[idx].add(Yg * Wg[:, :, None])

    mine = lax.dynamic_slice(partial, (my_x * T_LOC, 0), (T_LOC, D))
    theirs = lax.dynamic_slice(partial, ((1 - my_x) * T_LOC, 0), (T_LOC, D))
    return _exchange_add(mine, theirs, collective_id=2)


# baseline (device time: 509632 ns/iter reference)
import jax
import jax.numpy as jnp
from jax import lax
from jax.experimental import pallas as pl
from jax.experimental.pallas import tpu as pltpu

E = 16
E_LOC = 8
T = 2048
T_LOC = 1024
D = 1024
F = 4096
CAP = 384
FT = 1024

_MESH = pl.DeviceIdType.MESH if hasattr(pl, "DeviceIdType") else pltpu.DeviceIdType.MESH
_CompilerParams = getattr(pltpu, "CompilerParams", None) or pltpu.TPUCompilerParams


def _partner():
    return (1 - lax.axis_index("x"), lax.axis_index("y"), lax.axis_index("z"))


def _partner_barrier():
    barrier = pltpu.get_barrier_semaphore()
    pl.semaphore_signal(barrier, inc=1, device_id=_partner(), device_id_type=_MESH)
    pl.semaphore_wait(barrier, 1)


def _pairwise_exchange(arrays, collective_id):
    n = len(arrays)

    def body(*refs):
        ins = refs[:n]
        outs = refs[n : 2 * n]
        send_sems = refs[2 * n]
        recv_sems = refs[2 * n + 1]
        _partner_barrier()
        rdmas = []
        for k in range(n):
            r = pltpu.make_async_remote_copy(
                src_ref=ins[k],
                dst_ref=outs[k],
                send_sem=send_sems.at[k],
                recv_sem=recv_sems.at[k],
                device_id=_partner(),
                device_id_type=_MESH,
            )
            r.start()
            rdmas.append(r)
        for r in rdmas:
            r.wait()

    out = pl.pallas_call(
        body,
        out_shape=[jax.ShapeDtypeStruct(a.shape, a.dtype) for a in arrays],
        in_specs=[pl.BlockSpec(memory_space=pltpu.VMEM)] * n,
        out_specs=[pl.BlockSpec(memory_space=pltpu.VMEM)] * n,
        scratch_shapes=[
            pltpu.SemaphoreType.DMA((n,)),
            pltpu.SemaphoreType.DMA((n,)),
        ],
        compiler_params=_CompilerParams(collective_id=collective_id),
    )(*arrays)
    return out


def _exchange_add(mine, theirs, collective_id):

    def body(mine_ref, theirs_ref, out_ref, recv_buf, send_sem, recv_sem):
        _partner_barrier()
        r = pltpu.make_async_remote_copy(
            src_ref=theirs_ref,
            dst_ref=recv_buf,
            send_sem=send_sem,
            recv_sem=recv_sem,
            device_id=_partner(),
            device_id_type=_MESH,
        )
        r.start()
        r.wait()
        out_ref[...] = mine_ref[...] + recv_buf[...]

    return pl.pallas_call(
        body,
        out_shape=jax.ShapeDtypeStruct(mine.shape, mine.dtype),
        in_specs=[pl.BlockSpec(memory_space=pltpu.VMEM)] * 2,
        out_specs=pl.BlockSpec(memory_space=pltpu.VMEM),
        scratch_shapes=[
            pltpu.VMEM(mine.shape, mine.dtype),
            pltpu.SemaphoreType.DMA,
            pltpu.SemaphoreType.DMA,
        ],
        compiler_params=_CompilerParams(collective_id=collective_id),
    )(mine, theirs)


def _ffn(Xg, W1, W2):

    def body(xg_ref, w1_ref, w2_ref, yg_ref):
        @pl.when(pl.program_id(1) == 0)
        def _init():
            yg_ref[...] = jnp.zeros_like(yg_ref)

        h = jnp.dot(
            xg_ref[0],
            w1_ref[0].astype(jnp.bfloat16),
            preferred_element_type=jnp.float32,
        )
        h = jnp.maximum(h, 0.0).astype(jnp.bfloat16)
        yg_ref[0] += jnp.dot(
            h, w2_ref[0].astype(jnp.bfloat16), preferred_element_type=jnp.float32
        )

    return pl.pallas_call(
        body,
        grid=(E_LOC, F // FT),
        in_specs=[
            pl.BlockSpec((1, CAP, D), lambda e, f: (e, 0, 0)),
            pl.BlockSpec((1, D, FT), lambda e, f: (e, 0, f)),
            pl.BlockSpec((1, FT, D), lambda e, f: (e, f, 0)),
        ],
        out_specs=pl.BlockSpec((1, CAP, D), lambda e, f: (e, 0, 0)),
        out_shape=jax.ShapeDtypeStruct((E_LOC, CAP, D), jnp.float32),
        compiler_params=_CompilerParams(
            dimension_semantics=("arbitrary", "arbitrary")
        ),
    )(Xg, W1, W2)


def kernel(x, router, W1, W2):
    my_x = lax.axis_index("x")

    def cat(a, b, axis):
        return jnp.where(
            my_x == 0,
            jnp.concatenate([a, b], axis=axis),
            jnp.concatenate([b, a], axis=axis),
        )

    (router_other,) = _pairwise_exchange([router], collective_id=0)
    router_full = cat(router, router_other, axis=1)

    gates_loc = jnp.dot(x, router_full, precision=lax.Precision.HIGHEST)
    x_bf = x.astype(jnp.bfloat16)
    x_other, gates_other = _pairwise_exchange([x_bf, gates_loc], collective_id=1)
    x_full = cat(x_bf, x_other, axis=0)
    gates = cat(gates_loc, gates_other, axis=0)

    top_v, top_i = lax.top_k(gates, 2)
    w = jnp.exp(top_v - top_v[:, 0:1])
    w = w / w.sum(axis=1, keepdims=True)
    w_dense = jnp.zeros((T, E), jnp.float32).at[jnp.arange(T)[:, None], top_i].add(w)
    w_loc = lax.dynamic_slice(w_dense, (0, my_x * E_LOC), (T, E_LOC))

    chosen = w_loc > 0
    order = jnp.argsort(~chosen, axis=0, stable=True)
    idx = order[:CAP, :].T
    Xg = x_full[idx]
    Wg = jnp.take_along_axis(w_loc, idx.T, axis=0).T

    Yg = _ffn(Xg, W1, W2)
    partial = jnp.zeros((T, D), jnp.float32).at[idx].add(Yg * Wg[:, :, None])

    mine = lax.dynamic_slice(partial, (my_x * T_LOC, 0), (T_LOC, D))
    theirs = lax.dynamic_slice(partial, ((1 - my_x) * T_LOC, 0), (T_LOC, D))
    return _exchange_add(mine, theirs, collective_id=2)


# device time: 234754 ns/iter; 2.1709x vs baseline; 2.1709x over previous
import jax
import jax.numpy as jnp
from jax import lax
from jax.experimental import pallas as pl
from jax.experimental.pallas import tpu as pltpu

E = 16
E_LOC = 8
T = 2048
T_LOC = 1024
D = 1024
F = 4096
CAP = 384
FT = 1024

_MESH = pl.DeviceIdType.MESH if hasattr(pl, "DeviceIdType") else pltpu.DeviceIdType.MESH
_CompilerParams = getattr(pltpu, "CompilerParams", None) or pltpu.TPUCompilerParams


def _partner():
    return (1 - lax.axis_index("x"), lax.axis_index("y"), lax.axis_index("z"))


def _partner_barrier():
    barrier = pltpu.get_barrier_semaphore()
    pl.semaphore_signal(barrier, inc=1, device_id=_partner(), device_id_type=_MESH)
    pl.semaphore_wait(barrier, 1)


def _pairwise_exchange(arrays, collective_id):
    n = len(arrays)

    def body(*refs):
        ins = refs[:n]
        outs = refs[n : 2 * n]
        send_sems = refs[2 * n]
        recv_sems = refs[2 * n + 1]
        _partner_barrier()
        rdmas = []
        for k in range(n):
            r = pltpu.make_async_remote_copy(
                src_ref=ins[k],
                dst_ref=outs[k],
                send_sem=send_sems.at[k],
                recv_sem=recv_sems.at[k],
                device_id=_partner(),
                device_id_type=_MESH,
            )
            r.start()
            rdmas.append(r)
        for r in rdmas:
            r.wait()

    out = pl.pallas_call(
        body,
        out_shape=[jax.ShapeDtypeStruct(a.shape, a.dtype) for a in arrays],
        in_specs=[pl.BlockSpec(memory_space=pltpu.VMEM)] * n,
        out_specs=[pl.BlockSpec(memory_space=pltpu.VMEM)] * n,
        scratch_shapes=[
            pltpu.SemaphoreType.DMA((n,)),
            pltpu.SemaphoreType.DMA((n,)),
        ],
        compiler_params=_CompilerParams(collective_id=collective_id),
    )(*arrays)
    return out


def _exchange_add(mine, theirs, collective_id):

    def body(mine_ref, theirs_ref, out_ref, recv_buf, send_sem, recv_sem):
        _partner_barrier()
        r = pltpu.make_async_remote_copy(
            src_ref=theirs_ref,
            dst_ref=recv_buf,
            send_sem=send_sem,
            recv_sem=recv_sem,
            device_id=_partner(),
            device_id_type=_MESH,
        )
        r.start()
        r.wait()
        out_ref[...] = mine_ref[...] + recv_buf[...]

    return pl.pallas_call(
        body,
        out_shape=jax.ShapeDtypeStruct(mine.shape, mine.dtype),
        in_specs=[pl.BlockSpec(memory_space=pltpu.VMEM)] * 2,
        out_specs=pl.BlockSpec(memory_space=pltpu.VMEM),
        scratch_shapes=[
            pltpu.VMEM(mine.shape, mine.dtype),
            pltpu.SemaphoreType.DMA,
            pltpu.SemaphoreType.DMA,
        ],
        compiler_params=_CompilerParams(collective_id=collective_id),
    )(mine, theirs)


def _ffn(Xg, W1, W2):

    def body(xg_ref, w1_ref, w2_ref, yg_ref):
        @pl.when(pl.program_id(1) == 0)
        def _init():
            yg_ref[...] = jnp.zeros_like(yg_ref)

        h = jnp.dot(
            xg_ref[0],
            w1_ref[0].astype(jnp.bfloat16),
            preferred_element_type=jnp.float32,
        )
        h = jnp.maximum(h, 0.0).astype(jnp.bfloat16)
        yg_ref[0] += jnp.dot(
            h, w2_ref[0].astype(jnp.bfloat16), preferred_element_type=jnp.float32
        )

    return pl.pallas_call(
        body,
        grid=(E_LOC, F // FT),
        in_specs=[
            pl.BlockSpec((1, CAP, D), lambda e, f: (e, 0, 0)),
            pl.BlockSpec((1, D, FT), lambda e, f: (e, 0, f)),
            pl.BlockSpec((1, FT, D), lambda e, f: (e, f, 0)),
        ],
        out_specs=pl.BlockSpec((1, CAP, D), lambda e, f: (e, 0, 0)),
        out_shape=jax.ShapeDtypeStruct((E_LOC, CAP, D), jnp.float32),
        compiler_params=_CompilerParams(
            dimension_semantics=("arbitrary", "arbitrary")
        ),
    )(Xg, W1, W2)


def kernel(x, router, W1, W2):
    my_x = lax.axis_index("x")

    def cat(a, b, axis):
        return jnp.where(
            my_x == 0,
            jnp.concatenate([a, b], axis=axis),
            jnp.concatenate([b, a], axis=axis),
        )

    (router_other,) = _pairwise_exchange([router], collective_id=0)
    router_full = cat(router, router_other, axis=1)

    gates_loc = jnp.dot(x, router_full, precision=lax.Precision.HIGHEST)
    x_bf = x.astype(jnp.bfloat16)
    x_other, gates_other = _pairwise_exchange([x_bf, gates_loc], collective_id=1)
    x_full = cat(x_bf, x_other, axis=0)
    gates = cat(gates_loc, gates_other, axis=0)

    top_v, top_i = lax.top_k(gates, 2)
    w = jnp.exp(top_v - top_v[:, 0:1])
    w = w / w.sum(axis=1, keepdims=True)
    onehot = top_i[:, :, None] == jnp.arange(E)[None, None, :]
    w_dense = jnp.sum(onehot * w[:, :, None], axis=1)
    w_loc = lax.dynamic_slice(w_dense, (0, my_x * E_LOC), (T, E_LOC))

    chosen = w_loc > 0
    order = jnp.argsort(~chosen, axis=0, stable=True)
    idx = order[:CAP, :].T
    sel = (idx[:, :, None] == jnp.arange(T)[None, None, :]).astype(jnp.bfloat16)
    Xg = jnp.einsum("ect,td->ecd", sel, x_full,
                    preferred_element_type=jnp.bfloat16)
    Wg = jnp.take_along_axis(w_loc, idx.T, axis=0).T

    Yg = _ffn(Xg, W1, W2)
    contrib = (Yg * Wg[:, :, None]).astype(jnp.bfloat16)
    partial = jnp.einsum("ect,ecd->td", sel, contrib,
                         preferred_element_type=jnp.float32)

    mine = lax.dynamic_slice(partial, (my_x * T_LOC, 0), (T_LOC, D))
    theirs = lax.dynamic_slice(partial, ((1 - my_x) * T_LOC, 0), (T_LOC, D))
    return _exchange_add(mine, theirs, collective_id=2)


# device time: 229363 ns/iter; 2.2219x vs baseline; 1.0235x over previous
import jax
import jax.numpy as jnp
from jax import lax
from jax.experimental import pallas as pl
from jax.experimental.pallas import tpu as pltpu

E = 16
E_LOC = 8
T = 2048
T_LOC = 1024
D = 1024
F = 4096
CAP = 384
FT = 1024

_MESH = pl.DeviceIdType.MESH if hasattr(pl, "DeviceIdType") else pltpu.DeviceIdType.MESH
_CompilerParams = getattr(pltpu, "CompilerParams", None) or pltpu.TPUCompilerParams


def _partner():
    return (1 - lax.axis_index("x"), lax.axis_index("y"), lax.axis_index("z"))


def _partner_barrier():
    barrier = pltpu.get_barrier_semaphore()
    pl.semaphore_signal(barrier, inc=1, device_id=_partner(), device_id_type=_MESH)
    pl.semaphore_wait(barrier, 1)


def _pairwise_exchange(arrays, collective_id):
    n = len(arrays)

    def body(*refs):
        ins = refs[:n]
        outs = refs[n : 2 * n]
        send_sems = refs[2 * n]
        recv_sems = refs[2 * n + 1]
        _partner_barrier()
        rdmas = []
        for k in range(n):
            r = pltpu.make_async_remote_copy(
                src_ref=ins[k],
                dst_ref=outs[k],
                send_sem=send_sems.at[k],
                recv_sem=recv_sems.at[k],
                device_id=_partner(),
                device_id_type=_MESH,
            )
            r.start()
            rdmas.append(r)
        for r in rdmas:
            r.wait()

    out = pl.pallas_call(
        body,
        out_shape=[jax.ShapeDtypeStruct(a.shape, a.dtype) for a in arrays],
        in_specs=[pl.BlockSpec(memory_space=pltpu.VMEM)] * n,
        out_specs=[pl.BlockSpec(memory_space=pltpu.VMEM)] * n,
        scratch_shapes=[
            pltpu.SemaphoreType.DMA((n,)),
            pltpu.SemaphoreType.DMA((n,)),
        ],
        compiler_params=_CompilerParams(collective_id=collective_id),
    )(*arrays)
    return out


def _exchange_add(mine, theirs, collective_id):

    def body(mine_ref, theirs_ref, out_ref, recv_buf, send_sem, recv_sem):
        _partner_barrier()
        r = pltpu.make_async_remote_copy(
            src_ref=theirs_ref,
            dst_ref=recv_buf,
            send_sem=send_sem,
            recv_sem=recv_sem,
            device_id=_partner(),
            device_id_type=_MESH,
        )
        r.start()
        r.wait()
        out_ref[...] = mine_ref[...] + recv_buf[...]

    return pl.pallas_call(
        body,
        out_shape=jax.ShapeDtypeStruct(mine.shape, mine.dtype),
        in_specs=[pl.BlockSpec(memory_space=pltpu.VMEM)] * 2,
        out_specs=pl.BlockSpec(memory_space=pltpu.VMEM),
        scratch_shapes=[
            pltpu.VMEM(mine.shape, mine.dtype),
            pltpu.SemaphoreType.DMA,
            pltpu.SemaphoreType.DMA,
        ],
        compiler_params=_CompilerParams(collective_id=collective_id),
    )(mine, theirs)


def _ffn(Xg, W1, W2):

    def body(xg_ref, w1_ref, w2_ref, yg_ref):
        @pl.when(pl.program_id(1) == 0)
        def _init():
            yg_ref[...] = jnp.zeros_like(yg_ref)

        h = jnp.dot(
            xg_ref[0],
            w1_ref[0].astype(jnp.bfloat16),
            preferred_element_type=jnp.float32,
        )
        h = jnp.maximum(h, 0.0).astype(jnp.bfloat16)
        yg_ref[0] += jnp.dot(
            h, w2_ref[0].astype(jnp.bfloat16), preferred_element_type=jnp.float32
        )

    return pl.pallas_call(
        body,
        grid=(E_LOC, F // FT),
        in_specs=[
            pl.BlockSpec((1, CAP, D), lambda e, f: (e, 0, 0)),
            pl.BlockSpec((1, D, FT), lambda e, f: (e, 0, f)),
            pl.BlockSpec((1, FT, D), lambda e, f: (e, f, 0)),
        ],
        out_specs=pl.BlockSpec((1, CAP, D), lambda e, f: (e, 0, 0)),
        out_shape=jax.ShapeDtypeStruct((E_LOC, CAP, D), jnp.float32),
        compiler_params=_CompilerParams(
            dimension_semantics=("arbitrary", "arbitrary")
        ),
    )(Xg, W1, W2)


def kernel(x, router, W1, W2):
    my_x = lax.axis_index("x")

    def cat(a, b, axis):
        return jnp.where(
            my_x == 0,
            jnp.concatenate([a, b], axis=axis),
            jnp.concatenate([b, a], axis=axis),
        )

    (router_other,) = _pairwise_exchange([router], collective_id=0)
    router_full = cat(router, router_other, axis=1)

    gates_loc = jnp.dot(x, router_full, precision=lax.Precision.HIGHEST)
    x_bf = x.astype(jnp.bfloat16)
    x_other, gates_other = _pairwise_exchange([x_bf, gates_loc], collective_id=1)
    x_full = cat(x_bf, x_other, axis=0)
    gates = cat(gates_loc, gates_other, axis=0)

    top_v, top_i = lax.top_k(gates, 2)
    w = jnp.exp(top_v - top_v[:, 0:1])
    w = w / w.sum(axis=1, keepdims=True)
    onehot = top_i[:, :, None] == jnp.arange(E)[None, None, :]
    w_dense = jnp.sum(onehot * w[:, :, None], axis=1)
    w_loc = lax.dynamic_slice(w_dense, (0, my_x * E_LOC), (T, E_LOC))

    chosen = w_loc > 0
    order = jnp.argsort(~chosen, axis=0, stable=True)
    idx = order[:CAP, :].T
    sel = (idx[:, :, None] == jnp.arange(T)[None, None, :]).astype(jnp.bfloat16)
    Xg = jnp.einsum("ect,td->ecd", sel, x_full,
                    preferred_element_type=jnp.bfloat16)
    Wg = jnp.einsum("ect,et->ec", sel.astype(jnp.float32), w_loc.T)

    Yg = _ffn(Xg, W1, W2)
    contrib = (Yg * Wg[:, :, None]).astype(jnp.bfloat16)
    partial = jnp.einsum("ect,ecd->td", sel, contrib,
                         preferred_element_type=jnp.float32)

    mine = lax.dynamic_slice(partial, (my_x * T_LOC, 0), (T_LOC, D))
    theirs = lax.dynamic_slice(partial, ((1 - my_x) * T_LOC, 0), (T_LOC, D))
    return _exchange_add(mine, theirs, collective_id=2)


# device time: 205727 ns/iter; 2.4772x vs baseline; 1.1149x over previous
import jax
import jax.numpy as jnp
from jax import lax
from jax.experimental import pallas as pl
from jax.experimental.pallas import tpu as pltpu

E = 16
E_LOC = 8
T = 2048
T_LOC = 1024
D = 1024
F = 4096
CAP = 384
FT = 1024

_MESH = pl.DeviceIdType.MESH if hasattr(pl, "DeviceIdType") else pltpu.DeviceIdType.MESH
_CompilerParams = getattr(pltpu, "CompilerParams", None) or pltpu.TPUCompilerParams


def _partner():
    return (1 - lax.axis_index("x"), lax.axis_index("y"), lax.axis_index("z"))


def _partner_barrier():
    barrier = pltpu.get_barrier_semaphore()
    pl.semaphore_signal(barrier, inc=1, device_id=_partner(), device_id_type=_MESH)
    pl.semaphore_wait(barrier, 1)


def _pairwise_exchange(arrays, collective_id):
    n = len(arrays)

    def body(*refs):
        ins = refs[:n]
        outs = refs[n : 2 * n]
        send_sems = refs[2 * n]
        recv_sems = refs[2 * n + 1]
        _partner_barrier()
        rdmas = []
        for k in range(n):
            r = pltpu.make_async_remote_copy(
                src_ref=ins[k],
                dst_ref=outs[k],
                send_sem=send_sems.at[k],
                recv_sem=recv_sems.at[k],
                device_id=_partner(),
                device_id_type=_MESH,
            )
            r.start()
            rdmas.append(r)
        for r in rdmas:
            r.wait()

    out = pl.pallas_call(
        body,
        out_shape=[jax.ShapeDtypeStruct(a.shape, a.dtype) for a in arrays],
        in_specs=[pl.BlockSpec(memory_space=pltpu.VMEM)] * n,
        out_specs=[pl.BlockSpec(memory_space=pltpu.VMEM)] * n,
        scratch_shapes=[
            pltpu.SemaphoreType.DMA((n,)),
            pltpu.SemaphoreType.DMA((n,)),
        ],
        compiler_params=_CompilerParams(collective_id=collective_id),
    )(*arrays)
    return out


def _exchange_add(mine, theirs, collective_id):

    def body(mine_ref, theirs_ref, out_ref, recv_buf, send_sem, recv_sem):
        _partner_barrier()
        r = pltpu.make_async_remote_copy(
            src_ref=theirs_ref,
            dst_ref=recv_buf,
            send_sem=send_sem,
            recv_sem=recv_sem,
            device_id=_partner(),
            device_id_type=_MESH,
        )
        r.start()
        r.wait()
        out_ref[...] = mine_ref[...] + recv_buf[...].astype(mine_ref.dtype)

    return pl.pallas_call(
        body,
        out_shape=jax.ShapeDtypeStruct(mine.shape, mine.dtype),
        in_specs=[pl.BlockSpec(memory_space=pltpu.VMEM)] * 2,
        out_specs=pl.BlockSpec(memory_space=pltpu.VMEM),
        scratch_shapes=[
            pltpu.VMEM(theirs.shape, theirs.dtype),
            pltpu.SemaphoreType.DMA,
            pltpu.SemaphoreType.DMA,
        ],
        compiler_params=_CompilerParams(collective_id=collective_id),
    )(mine, theirs)


def _ffn(Xg, W1, W2):

    def body(xg_ref, w1_ref, w2_ref, yg_ref):
        @pl.when(pl.program_id(1) == 0)
        def _init():
            yg_ref[...] = jnp.zeros_like(yg_ref)

        h = jnp.dot(
            xg_ref[0],
            w1_ref[0].astype(jnp.bfloat16),
            preferred_element_type=jnp.float32,
        )
        h = jnp.maximum(h, 0.0).astype(jnp.bfloat16)
        yg_ref[0] += jnp.dot(
            h, w2_ref[0].astype(jnp.bfloat16), preferred_element_type=jnp.float32
        )

    return pl.pallas_call(
        body,
        grid=(E_LOC, F // FT),
        in_specs=[
            pl.BlockSpec((1, CAP, D), lambda e, f: (e, 0, 0)),
            pl.BlockSpec((1, D, FT), lambda e, f: (e, 0, f)),
            pl.BlockSpec((1, FT, D), lambda e, f: (e, f, 0)),
        ],
        out_specs=pl.BlockSpec((1, CAP, D), lambda e, f: (e, 0, 0)),
        out_shape=jax.ShapeDtypeStruct((E_LOC, CAP, D), jnp.float32),
        compiler_params=_CompilerParams(
            dimension_semantics=("arbitrary", "arbitrary")
        ),
    )(Xg, W1, W2)


def kernel(x, router, W1, W2):
    my_x = lax.axis_index("x")

    def cat(a, b, axis):
        return jnp.where(
            my_x == 0,
            jnp.concatenate([a, b], axis=axis),
            jnp.concatenate([b, a], axis=axis),
        )

    x_bf = x.astype(jnp.bfloat16)
    router_other, x_other = _pairwise_exchange([router, x_bf], collective_id=0)
    router_full = cat(router, router_other, axis=1)
    x_full = cat(x_bf, x_other, axis=0)

    gates_loc = jnp.dot(x, router_full, precision=lax.Precision.HIGHEST)
    (gates_other,) = _pairwise_exchange([gates_loc], collective_id=1)
    gates = cat(gates_loc, gates_other, axis=0)

    top_v, top_i = lax.top_k(gates, 2)
    w = jnp.exp(top_v - top_v[:, 0:1])
    w = w / w.sum(axis=1, keepdims=True)
    onehot = top_i[:, :, None] == jnp.arange(E)[None, None, :]
    w_dense = jnp.sum(onehot * w[:, :, None], axis=1)
    w_loc = lax.dynamic_slice(w_dense, (0, my_x * E_LOC), (T, E_LOC))

    chosen = w_loc > 0
    order = jnp.argsort(~chosen, axis=0, stable=True)
    idx = order[:CAP, :].T
    sel = (idx[:, :, None] == jnp.arange(T)[None, None, :]).astype(jnp.bfloat16)
    Xg = jnp.einsum("ect,td->ecd", sel, x_full,
                    preferred_element_type=jnp.bfloat16)
    Wg = jnp.einsum("ect,et->ec", sel.astype(jnp.float32), w_loc.T)

    Yg = _ffn(Xg, W1, W2)
    contrib = (Yg * Wg[:, :, None]).astype(jnp.bfloat16)
    partial = jnp.einsum("ect,ecd->td", sel, contrib,
                         preferred_element_type=jnp.float32)

    mine = lax.dynamic_slice(partial, (my_x * T_LOC, 0), (T_LOC, D))
    theirs = lax.dynamic_slice(partial, ((1 - my_x) * T_LOC, 0), (T_LOC, D))
    return _exchange_add(mine, theirs.astype(jnp.bfloat16), collective_id=2)
